# baseline (speedup 1.0000x reference)
import functools

import jax
import jax.numpy as jnp
from jax import lax
from jax.experimental import pallas as pl
from jax.experimental.pallas import tpu as pltpu
from jax.experimental.pallas import tpu_sc as plsc

N = 10000
E = 320000
NPS = 10112
EP = 327680
K = 128
NC = 2
NS = 16
NW = NC * NS
E_PER_W = EP // NW
NCH = E_PER_W // K
NBUF = 2
GROUPS = NCH // NBUF
RPT = NPS // NS


def _spmm_sc(tlo, thi, src3, dst3, zeros, dh, local, count_hi=False):
    mesh = plsc.VectorSubcoreMesh(core_axis_name="c", subcore_axis_name="s")

    scratch = [
        pltpu.VMEM((NBUF, K), jnp.int32),
        pltpu.VMEM((NBUF, K), jnp.int32),
        pltpu.VMEM((NBUF, K, dh), jnp.float32),
        pltpu.VMEM_SHARED((NPS, dh), jnp.float32),
    ]
    if local:
        scratch.append(pltpu.VMEM_SHARED((NPS, dh), jnp.float32))
    scratch += [pltpu.SemaphoreType.DMA] * (4 * NBUF)

    @functools.partial(
        pl.kernel,
        out_type=jax.ShapeDtypeStruct((2, NC, NPS, dh), jnp.float32),
        mesh=mesh,
        scratch_types=scratch,
        compiler_params=pltpu.CompilerParams(use_tc_tiling_on_sc=False),
    )
    def spmm(tlo_hbm, thi_hbm, src_hbm, dst_hbm, zeros_hbm, out_hbm,
             srcs_v, dsts_v, rows_v, acc_sh, *rest):
        if local:
            tab_sh, *sems = rest
        else:
            sems = rest
        gsem = sems[0:NBUF]
        ssem = sems[NBUF:2 * NBUF]
        dsem = sems[2 * NBUF:3 * NBUF]
        rsem = sems[3 * NBUF:4 * NBUF]
        cid = lax.axis_index("c")
        sid = lax.axis_index("s")
        wid = sid * NC + cid
        r0 = sid * RPT

        def wait_gather(b):
            pltpu.make_async_copy(zeros_hbm.at[pl.ds(0, K)],
                                  rows_v.at[b], gsem[b]).wait()

        def wait_scatter(b):
            pltpu.make_async_copy(rows_v.at[b],
                                  acc_sh.at[pl.ds(0, K)], ssem[b]).wait()

        def wait_dst(b):
            pltpu.make_async_copy(dst_hbm.at[wid, 0],
                                  dsts_v.at[b], dsem[b]).wait()

        def wait_src(b):
            pltpu.make_async_copy(src_hbm.at[wid, 0],
                                  srcs_v.at[b], rsem[b]).wait()

        for half, tab_in in enumerate((tlo_hbm, thi_hbm)):
            const_rows = count_hi and half == 1
            if const_rows:
                tab = None
                for b in range(NBUF):
                    pltpu.sync_copy(tab_in, rows_v.at[b])
            elif local:
                pltpu.sync_copy(tab_in.at[pl.ds(r0, RPT)],
                                tab_sh.at[pl.ds(r0, RPT)])
                tab = tab_sh
            else:
                tab = tab_in
            pltpu.sync_copy(zeros_hbm.at[pl.ds(r0, RPT)],
                            acc_sh.at[pl.ds(r0, RPT)])
            plsc.subcore_barrier()

            for b in range(NBUF):
                pltpu.sync_copy(dst_hbm.at[wid, b], dsts_v.at[b])
                if not const_rows:
                    pltpu.sync_copy(src_hbm.at[wid, b], srcs_v.at[b])
                    pltpu.async_copy(tab.at[srcs_v.at[b]], rows_v.at[b],
                                     gsem[b])

            @pl.loop(0, GROUPS)
            def grp(g):
                for b in range(NBUF):
                    if not const_rows:
                        wait_gather(b)

                    @pl.when(g > 0)
                    def _():
                        wait_dst(b)
                    pltpu.async_copy(rows_v.at[b],
                                     acc_sh.at[dsts_v.at[b]],
                                     ssem[b], add=True)

                @pl.when(g < GROUPS - 1)
                def _():
                    for b in range(NBUF):
                        wait_scatter(b)
                        pltpu.async_copy(
                            dst_hbm.at[wid, (g + 1) * NBUF + b],
                            dsts_v.at[b], dsem[b])
                        if not const_rows:
                            pltpu.async_copy(
                                src_hbm.at[wid, (g + 1) * NBUF + b],
                                srcs_v.at[b], rsem[b])
                            wait_src(b)
                            pltpu.async_copy(tab.at[srcs_v.at[b]],
                                             rows_v.at[b], gsem[b])

            for b in range(NBUF):
                wait_scatter(b)
            plsc.subcore_barrier()
            pltpu.sync_copy(acc_sh.at[pl.ds(r0, RPT)],
                            out_hbm.at[half, cid, pl.ds(r0, RPT)])

    return spmm(tlo, thi, src3, dst3, zeros)


BLK = 632


def _dense_layer(h, gp, efap, wmt, wme, bm, wat, wan, ba):
    din = h.shape[1]
    dh = din // 2
    dm = wmt.shape[1]

    def body(h_ref, g_ref, efa_ref, wmt_ref, wme_ref, bm_ref,
             wat_ref, wan_ref, ba_ref, out_ref):
        g_lo = g_ref[0, 0] + g_ref[0, 1]
        g_hi = g_ref[1, 0] + g_ref[1, 1]
        efa = efa_ref[0, 0] + efa_ref[0, 1]
        cnt = (efa_ref[1, 0] + efa_ref[1, 1])[:, 0:1]
        inv = 1.0 / jnp.maximum(cnt, 1.0)
        s = (jnp.dot(g_lo, wmt_ref[:dh], preferred_element_type=jnp.float32)
             + jnp.dot(g_hi, wmt_ref[dh:],
                       preferred_element_type=jnp.float32)
             + jnp.dot(efa, wme_ref[...], preferred_element_type=jnp.float32)
             + cnt * bm_ref[...])
        hn = s * inv
        out = (jnp.dot(h_ref[...], wat_ref[...],
                       preferred_element_type=jnp.float32)
               + jnp.dot(hn, wan_ref[...], preferred_element_type=jnp.float32)
               + ba_ref[...])
        out_ref[...] = jnp.maximum(out, 0.0)

    grid = NPS // BLK
    return pl.pallas_call(
        body,
        grid=(grid,),
        in_specs=[
            pl.BlockSpec((BLK, din), lambda i: (i, 0)),
            pl.BlockSpec((2, NC, BLK, dh), lambda i: (0, 0, i, 0)),
            pl.BlockSpec((2, NC, BLK, 16), lambda i: (0, 0, i, 0)),
            pl.BlockSpec((din, dm), lambda i: (0, 0)),
            pl.BlockSpec((16, dm), lambda i: (0, 0)),
            pl.BlockSpec((1, dm), lambda i: (0, 0)),
            pl.BlockSpec((din, dm), lambda i: (0, 0)),
            pl.BlockSpec((dm, dm), lambda i: (0, 0)),
            pl.BlockSpec((1, dm), lambda i: (0, 0)),
        ],
        out_specs=pl.BlockSpec((BLK, dm), lambda i: (i, 0)),
        out_shape=jax.ShapeDtypeStruct((NPS, dm), jnp.float32),
    )(h, gp, efap, wmt, wme, bm, wat, wan, ba)


def _pad2(w, r, c):
    return jnp.pad(w, ((0, r - w.shape[0]), (0, c - w.shape[1])))


def kernel(nfeats, edge_index, efeats, Wm1, bm1, Wa1, ba1,
           Wm2, bm2, Wa2, ba2, Wm3, bm3, Wa3, ba3):
    f32 = jnp.float32
    h0 = jnp.pad(nfeats[:, 0, :], ((0, NPS - N), (0, 0)))
    src3 = jnp.pad(edge_index[0], (0, EP - E)).reshape(NW, NCH, K)
    dst3 = jnp.pad(edge_index[1], (0, EP - E),
                   constant_values=N).reshape(NW, NCH, K)
    ef_tab = efeats[:, 0, :]
    ones_k = jnp.zeros((K, 16), f32).at[:, 0].set(1.0)
    eidx3 = jnp.pad(jnp.arange(E, dtype=jnp.int32),
                    (0, EP - E)).reshape(NW, NCH, K)

    z80 = jnp.zeros((NPS, 80), f32)
    z64 = jnp.zeros((NPS, 64), f32)
    z16 = jnp.zeros((NPS, 16), f32)

    efap = _spmm_sc(ef_tab, ones_k, eidx3, dst3, z16, 16, local=False,
                    count_hi=True)

    DH, DIN, DOUT, DHP = 152, 128, 128, 160
    g1 = _spmm_sc(h0[:, :64], h0[:, 64:], src3, dst3, z64, 64, local=True)
    h1 = _dense_layer(
        h0, g1, efap,
        _pad2(Wm1[:DIN], DIN, DHP), _pad2(Wm1[DIN:], 16, DHP),
        _pad2(bm1[None, :], 1, DHP),
        _pad2(Wa1[:DIN], DIN, DHP), _pad2(Wa1[DIN:], DHP, DHP),
        _pad2(ba1[None, :], 1, DHP))

    g2 = _spmm_sc(h1[:, :80], h1[:, 80:], src3, dst3, z80, 80, local=True)
    h2 = _dense_layer(
        h1, g2, efap,
        _pad2(Wm2[:DH], DHP, DHP), _pad2(Wm2[DH:], 16, DHP),
        _pad2(bm2[None, :], 1, DHP),
        _pad2(Wa2[:DH], DHP, DHP), _pad2(Wa2[DH:], DHP, DHP),
        _pad2(ba2[None, :], 1, DHP))

    g3 = _spmm_sc(h2[:, :80], h2[:, 80:], src3, dst3, z80, 80, local=True)
    h3 = _dense_layer(
        h2, g3, efap,
        _pad2(Wm3[:DH], DHP, DOUT), _pad2(Wm3[DH:], 16, DOUT),
        bm3[None, :],
        _pad2(Wa3[:DH], DHP, DOUT), _pad2(Wa3[DH:], DOUT, DOUT),
        ba3[None, :])

    return h3[:N]


# speedup vs baseline: 1.0383x; 1.0383x over previous
import functools

import jax
import jax.numpy as jnp
from jax import lax
from jax.experimental import pallas as pl
from jax.experimental.pallas import tpu as pltpu
from jax.experimental.pallas import tpu_sc as plsc

N = 10000
E = 320000
NPS = 10112
EP = 327680
K = 80
NC = 2
NS = 16
NW = NC * NS
E_PER_W = EP // NW
NCH = E_PER_W // K
NBUF = 2
GROUPS = NCH // NBUF
RPT = NPS // NS


def _spmm_sc(tlo, thi, src3, dst3, zeros, dh, local, count_hi=False):
    mesh = plsc.VectorSubcoreMesh(core_axis_name="c", subcore_axis_name="s")

    scratch = [
        pltpu.VMEM((NCH, K), jnp.int32),
        pltpu.VMEM((NBUF, K), jnp.int32),
        pltpu.VMEM((NBUF, K, dh), jnp.float32),
        pltpu.VMEM_SHARED((NPS, dh), jnp.float32),
    ]
    if local:
        scratch.append(pltpu.VMEM_SHARED((NPS, dh), jnp.float32))
    scratch += [pltpu.SemaphoreType.DMA] * (3 * NBUF)

    @functools.partial(
        pl.kernel,
        out_type=jax.ShapeDtypeStruct((2, NC, NPS, dh), jnp.float32),
        mesh=mesh,
        scratch_types=scratch,
        compiler_params=pltpu.CompilerParams(use_tc_tiling_on_sc=False),
    )
    def spmm(tlo_hbm, thi_hbm, src_hbm, dst_hbm, zeros_hbm, out_hbm,
             srcs_v, dsts_v, rows_v, acc_sh, *rest):
        if local:
            tab_sh, *sems = rest
        else:
            sems = rest
        gsem = sems[0:NBUF]
        ssem = sems[NBUF:2 * NBUF]
        dsem = sems[2 * NBUF:3 * NBUF]
        cid = lax.axis_index("c")
        sid = lax.axis_index("s")
        wid = sid * NC + cid
        r0 = sid * RPT
        pltpu.sync_copy(src_hbm.at[wid], srcs_v)

        def wait_gather(b):
            pltpu.make_async_copy(zeros_hbm.at[pl.ds(0, K)],
                                  rows_v.at[b], gsem[b]).wait()

        def wait_scatter(b):
            pltpu.make_async_copy(rows_v.at[b],
                                  acc_sh.at[pl.ds(0, K)], ssem[b]).wait()

        def wait_dst(b):
            pltpu.make_async_copy(dst_hbm.at[wid, 0],
                                  dsts_v.at[b], dsem[b]).wait()

        for half, tab_in in enumerate((tlo_hbm, thi_hbm)):
            const_rows = count_hi and half == 1
            if const_rows:
                tab = None
                for b in range(NBUF):
                    pltpu.sync_copy(tab_in, rows_v.at[b])
            elif local:
                pltpu.sync_copy(tab_in.at[pl.ds(r0, RPT)],
                                tab_sh.at[pl.ds(r0, RPT)])
                tab = tab_sh
            else:
                tab = tab_in
            pltpu.sync_copy(zeros_hbm.at[pl.ds(r0, RPT)],
                            acc_sh.at[pl.ds(r0, RPT)])
            plsc.subcore_barrier()

            for b in range(NBUF):
                pltpu.sync_copy(dst_hbm.at[wid, b], dsts_v.at[b])
                if not const_rows:
                    pltpu.async_copy(tab.at[srcs_v.at[b]], rows_v.at[b],
                                     gsem[b])

            @pl.loop(0, GROUPS)
            def grp(g):
                for b in range(NBUF):
                    if not const_rows:
                        wait_gather(b)

                    @pl.when(g > 0)
                    def _():
                        wait_dst(b)
                    pltpu.async_copy(rows_v.at[b],
                                     acc_sh.at[dsts_v.at[b]],
                                     ssem[b], add=True)

                @pl.when(g < GROUPS - 1)
                def _():
                    for b in range(NBUF):
                        wait_scatter(b)
                        pltpu.async_copy(
                            dst_hbm.at[wid, (g + 1) * NBUF + b],
                            dsts_v.at[b], dsem[b])
                        if not const_rows:
                            pltpu.async_copy(
                                tab.at[srcs_v.at[(g + 1) * NBUF + b]],
                                rows_v.at[b], gsem[b])

            for b in range(NBUF):
                wait_scatter(b)
            plsc.subcore_barrier()
            pltpu.sync_copy(acc_sh.at[pl.ds(r0, RPT)],
                            out_hbm.at[half, cid, pl.ds(r0, RPT)])

    return spmm(tlo, thi, src3, dst3, zeros)


BLK = 632


def _dense_layer(h, gp, efap, wmt, wme, bm, wat, wan, ba):
    din = h.shape[1]
    dh = din // 2
    dm = wmt.shape[1]

    def body(h_ref, g_ref, efa_ref, wmt_ref, wme_ref, bm_ref,
             wat_ref, wan_ref, ba_ref, out_ref):
        g_lo = g_ref[0, 0] + g_ref[0, 1]
        g_hi = g_ref[1, 0] + g_ref[1, 1]
        efa = efa_ref[0, 0] + efa_ref[0, 1]
        cnt = (efa_ref[1, 0] + efa_ref[1, 1])[:, 0:1]
        inv = 1.0 / jnp.maximum(cnt, 1.0)
        s = (jnp.dot(g_lo, wmt_ref[:dh], preferred_element_type=jnp.float32)
             + jnp.dot(g_hi, wmt_ref[dh:],
                       preferred_element_type=jnp.float32)
             + jnp.dot(efa, wme_ref[...], preferred_element_type=jnp.float32)
             + cnt * bm_ref[...])
        hn = s * inv
        out = (jnp.dot(h_ref[...], wat_ref[...],
                       preferred_element_type=jnp.float32)
               + jnp.dot(hn, wan_ref[...], preferred_element_type=jnp.float32)
               + ba_ref[...])
        out_ref[...] = jnp.maximum(out, 0.0)

    grid = NPS // BLK
    return pl.pallas_call(
        body,
        grid=(grid,),
        in_specs=[
            pl.BlockSpec((BLK, din), lambda i: (i, 0)),
            pl.BlockSpec((2, NC, BLK, dh), lambda i: (0, 0, i, 0)),
            pl.BlockSpec((2, NC, BLK, 16), lambda i: (0, 0, i, 0)),
            pl.BlockSpec((din, dm), lambda i: (0, 0)),
            pl.BlockSpec((16, dm), lambda i: (0, 0)),
            pl.BlockSpec((1, dm), lambda i: (0, 0)),
            pl.BlockSpec((din, dm), lambda i: (0, 0)),
            pl.BlockSpec((dm, dm), lambda i: (0, 0)),
            pl.BlockSpec((1, dm), lambda i: (0, 0)),
        ],
        out_specs=pl.BlockSpec((BLK, dm), lambda i: (i, 0)),
        out_shape=jax.ShapeDtypeStruct((NPS, dm), jnp.float32),
    )(h, gp, efap, wmt, wme, bm, wat, wan, ba)


def _pad2(w, r, c):
    return jnp.pad(w, ((0, r - w.shape[0]), (0, c - w.shape[1])))


def kernel(nfeats, edge_index, efeats, Wm1, bm1, Wa1, ba1,
           Wm2, bm2, Wa2, ba2, Wm3, bm3, Wa3, ba3):
    f32 = jnp.float32
    h0 = jnp.pad(nfeats[:, 0, :], ((0, NPS - N), (0, 0)))
    src3 = jnp.pad(edge_index[0], (0, EP - E)).reshape(NW, NCH, K)
    dst3 = jnp.pad(edge_index[1], (0, EP - E),
                   constant_values=N).reshape(NW, NCH, K)
    ef_tab = efeats[:, 0, :]
    ones_k = jnp.zeros((K, 16), f32).at[:, 0].set(1.0)
    eidx3 = jnp.pad(jnp.arange(E, dtype=jnp.int32),
                    (0, EP - E)).reshape(NW, NCH, K)

    z80 = jnp.zeros((NPS, 80), f32)
    z64 = jnp.zeros((NPS, 64), f32)
    z16 = jnp.zeros((NPS, 16), f32)

    efap = _spmm_sc(ef_tab, ones_k, eidx3, dst3, z16, 16, local=False,
                    count_hi=True)

    DH, DIN, DOUT, DHP = 152, 128, 128, 160
    g1 = _spmm_sc(h0[:, :64], h0[:, 64:], src3, dst3, z64, 64, local=True)
    h1 = _dense_layer(
        h0, g1, efap,
        _pad2(Wm1[:DIN], DIN, DHP), _pad2(Wm1[DIN:], 16, DHP),
        _pad2(bm1[None, :], 1, DHP),
        _pad2(Wa1[:DIN], DIN, DHP), _pad2(Wa1[DIN:], DHP, DHP),
        _pad2(ba1[None, :], 1, DHP))

    g2 = _spmm_sc(h1[:, :80], h1[:, 80:], src3, dst3, z80, 80, local=True)
    h2 = _dense_layer(
        h1, g2, efap,
        _pad2(Wm2[:DH], DHP, DHP), _pad2(Wm2[DH:], 16, DHP),
        _pad2(bm2[None, :], 1, DHP),
        _pad2(Wa2[:DH], DHP, DHP), _pad2(Wa2[DH:], DHP, DHP),
        _pad2(ba2[None, :], 1, DHP))

    g3 = _spmm_sc(h2[:, :80], h2[:, 80:], src3, dst3, z80, 80, local=True)
    h3 = _dense_layer(
        h2, g3, efap,
        _pad2(Wm3[:DH], DHP, DOUT), _pad2(Wm3[DH:], 16, DOUT),
        bm3[None, :],
        _pad2(Wa3[:DH], DHP, DOUT), _pad2(Wa3[DH:], DOUT, DOUT),
        ba3[None, :])

    return h3[:N]
